# baseline (speedup 1.0000x reference)
import functools

import jax
import jax.numpy as jnp
from jax import lax
from jax.experimental import pallas as pl
from jax.experimental.pallas import tpu as pltpu
from jax.experimental.pallas import tpu_sc as plsc


@functools.lru_cache(maxsize=None)
def _build(V, D, N, S):
    info = plsc.get_sparse_core_info()
    NC, NS, L = info.num_cores, info.num_subcores, info.num_lanes
    NW = NC * NS
    assert N % NW == 0
    n_per_w = N // NW
    C = 128
    NBUF = 6
    G = 3
    assert n_per_w % C == 0 and C % S == 0 and D % L == 0
    steps = n_per_w // C
    assert NBUF - G >= 2 or NBUF == 2 * G
    assert steps >= 2 * NBUF
    mesh = plsc.VectorSubcoreMesh(core_axis_name="c", subcore_axis_name="s")

    @functools.partial(
        pl.kernel,
        mesh=mesh,
        out_type=jax.ShapeDtypeStruct((N, D), jnp.float32),
        scratch_types=[
            pltpu.VMEM((n_per_w,), jnp.int32),
            pltpu.VMEM((S, D), jnp.float32),
            pltpu.VMEM((NBUF, C, D), jnp.float32),
        ]
        + [pltpu.SemaphoreType.DMA] * (2 * NBUF),
    )
    def k(ids_hbm, table_hbm, pos_hbm, out_hbm, idx_v, pos_v, buf_v, *sems):
        gsem = sems[0:NBUF]
        osem = sems[NBUF:2 * NBUF]
        wid = lax.axis_index("s") * NC + lax.axis_index("c")
        base = wid * n_per_w
        pltpu.sync_copy(ids_hbm.at[pl.ds(base, n_per_w)], idx_v)
        pltpu.sync_copy(pos_hbm, pos_v)

        def start_gather(t, b):
            pltpu.async_copy(
                table_hbm.at[idx_v.at[pl.ds(t * C, C)]], buf_v.at[b], gsem[b]
            )

        def wait_gather(b):
            pltpu.make_async_copy(
                table_hbm.at[idx_v.at[pl.ds(0, C)]], buf_v.at[b], gsem[b]
            ).wait()

        def start_out(t, b):
            pltpu.async_copy(
                buf_v.at[b], out_hbm.at[pl.ds(base + t * C, C)], osem[b]
            )

        def wait_out(b):
            pltpu.make_async_copy(
                buf_v.at[b], out_hbm.at[pl.ds(0, C)], osem[b]
            ).wait()

        def add_chunk(b):
            in_b = buf_v.at[b]

            def body(s, carry):
                psl = [pos_v[s, pl.ds(cc * L, L)] for cc in range(D // L)]
                for rep in range(C // S):
                    r = rep * S + s
                    for cc in range(D // L):
                        plsc.addupdate(in_b.at[r, pl.ds(cc * L, L)], psl[cc])
                return carry

            lax.fori_loop(0, S, body, 0, unroll=4)

        for b in range(G):
            start_gather(b, b)
        for t in range(G):
            wait_gather(t)
            add_chunk(t)
            start_out(t, t)
            start_gather(t + G, (t + G) % NBUF)

        n_steady = steps - 2 * G
        full_groups = n_steady // NBUF

        def group(gi, carry):
            T = G + gi * NBUF
            for j in range(NBUF):
                t = T + j
                s0, s2 = (G + j) % NBUF, (G + j + G) % NBUF
                wait_gather(s0)
                add_chunk(s0)
                start_out(t, s0)
                wait_out(s2)
                start_gather(t + G, s2)
            return carry

        lax.fori_loop(0, full_groups, group, 0, unroll=False)

        for t in range(G + full_groups * NBUF, steps - G):
            s0, s2 = t % NBUF, (t + G) % NBUF
            wait_gather(s0)
            add_chunk(s0)
            start_out(t, s0)
            wait_out(s2)
            start_gather(t + G, s2)

        for t in range(steps - G, steps):
            s0 = t % NBUF
            wait_gather(s0)
            add_chunk(s0)
            start_out(t, s0)
        for t in range(steps - NBUF, steps):
            wait_out(t % NBUF)

    return k


def kernel(input_ids, token_embedding, position_embedding):
    B, S = input_ids.shape
    V, D = token_embedding.shape
    N = B * S
    k = _build(V, D, N, S)
    out = k(input_ids.reshape(N), token_embedding, position_embedding)
    return out.reshape(B, S, D)


# speedup vs baseline: 1.0315x; 1.0315x over previous
import functools

import jax
import jax.numpy as jnp
from jax import lax
from jax.experimental import pallas as pl
from jax.experimental.pallas import tpu as pltpu
from jax.experimental.pallas import tpu_sc as plsc


@functools.lru_cache(maxsize=None)
def _build(V, D, N, S):
    info = plsc.get_sparse_core_info()
    NC, NS, L = info.num_cores, info.num_subcores, info.num_lanes
    NW = NC * NS
    assert N % NW == 0
    n_per_w = N // NW
    C = 128
    NBUF = 6
    G = 3
    assert n_per_w % C == 0 and C % S == 0 and D % L == 0
    steps = n_per_w // C
    assert NBUF - G >= 2 or NBUF == 2 * G
    assert steps >= 2 * NBUF
    mesh = plsc.VectorSubcoreMesh(core_axis_name="c", subcore_axis_name="s")

    @functools.partial(
        pl.kernel,
        mesh=mesh,
        out_type=jax.ShapeDtypeStruct((N, D), jnp.float32),
        scratch_types=[
            pltpu.VMEM((n_per_w,), jnp.int32),
            pltpu.VMEM((S, D), jnp.float32),
            pltpu.VMEM((NBUF, C, D), jnp.float32),
        ]
        + [pltpu.SemaphoreType.DMA] * (2 * NBUF),
    )
    def k(ids_hbm, table_hbm, pos_hbm, out_hbm, idx_v, pos_v, buf_v, *sems):
        gsem = sems[0:NBUF]
        osem = sems[NBUF:2 * NBUF]
        wid = lax.axis_index("s") * NC + lax.axis_index("c")
        base = wid * n_per_w
        pltpu.sync_copy(ids_hbm.at[pl.ds(base, n_per_w)], idx_v)
        pltpu.sync_copy(pos_hbm, pos_v)

        def start_gather(t, b):
            pltpu.async_copy(
                table_hbm.at[idx_v.at[pl.ds(t * C, C)]], buf_v.at[b], gsem[b]
            )

        def wait_gather(b):
            pltpu.make_async_copy(
                table_hbm.at[idx_v.at[pl.ds(0, C)]], buf_v.at[b], gsem[b]
            ).wait()

        def start_out(t, b):
            pltpu.async_copy(
                buf_v.at[b], out_hbm.at[pl.ds(base + t * C, C)], osem[b]
            )

        def wait_out(b):
            pltpu.make_async_copy(
                buf_v.at[b], out_hbm.at[pl.ds(0, C)], osem[b]
            ).wait()

        def add_chunk(b):
            in_b = buf_v.at[b]

            def body(s, carry):
                psl = [pos_v[s, pl.ds(cc * L, L)] for cc in range(D // L)]
                for rep in range(C // S):
                    r = rep * S + s
                    for cc in range(D // L):
                        plsc.addupdate(in_b.at[r, pl.ds(cc * L, L)], psl[cc])
                return carry

            lax.fori_loop(0, S, body, 0, unroll=2)

        for b in range(G):
            start_gather(b, b)
        for t in range(G):
            wait_gather(t)
            add_chunk(t)
            start_out(t, t)
            start_gather(t + G, (t + G) % NBUF)

        n_steady = steps - 2 * G
        full_groups = n_steady // NBUF

        def group(gi, carry):
            T = G + gi * NBUF
            for j in range(NBUF):
                t = T + j
                s0, s2 = (G + j) % NBUF, (G + j + G) % NBUF
                wait_gather(s0)
                add_chunk(s0)
                start_out(t, s0)
                wait_out(s2)
                start_gather(t + G, s2)
            return carry

        lax.fori_loop(0, full_groups, group, 0, unroll=False)

        for t in range(G + full_groups * NBUF, steps - G):
            s0, s2 = t % NBUF, (t + G) % NBUF
            wait_gather(s0)
            add_chunk(s0)
            start_out(t, s0)
            wait_out(s2)
            start_gather(t + G, s2)

        for t in range(steps - G, steps):
            s0 = t % NBUF
            wait_gather(s0)
            add_chunk(s0)
            start_out(t, s0)
        for t in range(steps - NBUF, steps):
            wait_out(t % NBUF)

    return k


def kernel(input_ids, token_embedding, position_embedding):
    B, S = input_ids.shape
    V, D = token_embedding.shape
    N = B * S
    k = _build(V, D, N, S)
    out = k(input_ids.reshape(N), token_embedding, position_embedding)
    return out.reshape(B, S, D)
